# baseline (speedup 1.0000x reference)
import functools
import jax
import jax.numpy as jnp
from jax import lax
from jax.experimental import pallas as pl
from jax.experimental.pallas import tpu as pltpu
from jax.experimental.pallas import tpu_sc as plsc

_W = 128
_C = 96
_D = 48
_CCHUNKS = _C // 16
_PAD = _D
_NW = 8
_NK = 4
_NKB = (_D + _NK - 1) // _NK + 1
_PITCH = 17


def _valid_pairs(k):
    return [
        (i, j)
        for i in range(_NW)
        for j in range(_NK)
        if 0 <= _D + i - _NK * k - j < _D
    ]


def _body(l_hbm, r_hbm, out_hbm, l_v, rpad_v, out_v, sem):
    n_cores = 2
    n_sub = 16
    wid = lax.axis_index("s") * n_cores + lax.axis_index("c")
    n_workers = n_cores * n_sub
    nrows = l_hbm.shape[0]
    rows_per = nrows // n_workers

    zero16 = jnp.zeros((16,), jnp.float32)
    scale = jnp.float32(1.0 / _C)
    iota = lax.iota(jnp.int32, 16)
    xmask = {s: (iota & s) != 0 for s in (8, 4, 2, 1)}
    xperm = {s: iota ^ s for s in (8, 4, 2, 1)}

    def merge(a, b, s):
        sel_ab = jnp.where(xmask[s], b, a)
        sel_ba = jnp.where(xmask[s], a, b)
        return sel_ab + sel_ba.at[xperm[s]].get(mode="promise_in_bounds")

    half_i = jnp.right_shift(iota, 1)
    low_j = jnp.bitwise_and(iota, 1)
    patvec = half_i * (_D + 1) - low_j
    dbase = half_i - low_j
    bitrev = [int(f"{t:04b}"[::-1], 2) for t in range(16)]

    def zero_chunk(i, _):
        rpad_v[pl.ds(i * 16, 16)] = zero16
        return 0

    lax.fori_loop(0, _PAD * _C // 16, zero_chunk, 0)

    def emit_tile(lbase, rbase, obase, koff, valid, full):
        used_i = sorted({i for i, _ in valid})
        used_j = sorted({j for _, j in valid})
        accs = {p: zero16 for p in valid}
        for cb in range(_CCHUNKS):
            lv = {i: l_v[pl.ds(lbase + (_C * i + 16 * cb), 16)] for i in used_i}
            rv = {
                j: rpad_v[pl.ds(rbase + (_C * j + 16 * cb), 16)]
                for j in used_j
            }
            for (i, j) in valid:
                accs[(i, j)] = accs[(i, j)] + lv[i] * rv[j]
        for g in range(2):
            leaves = []
            any_live = False
            for t in range(16):
                o = bitrev[t]
                p = (o >> 1, 2 * g + (o & 1))
                if p in accs:
                    leaves.append(accs[p])
                    any_live = True
                else:
                    leaves.append(zero16)
            if not any_live:
                continue
            vs = leaves
            for s in (8, 4, 2, 1):
                vs = [merge(vs[2 * m], vs[2 * m + 1], s) for m in range(len(vs) // 2)]
            tot = vs[0] * scale
            idx = patvec + (obase - 2 * g)
            if full:
                plsc.store_scatter(out_v, [idx], tot)
            else:
                dvec = dbase + (koff - 2 * g)
                mask = (dvec >= 0) & (dvec < _D)
                plsc.store_scatter(out_v, [idx], tot, mask=mask)

    def do_row(r, _):
        row = wid * rows_per + r
        pltpu.sync_copy(l_hbm.at[row], l_v)
        pltpu.sync_copy(r_hbm.at[row], rpad_v.at[pl.ds(_PAD * _C, _W * _C)])

        all_pairs = [(i, j) for i in range(_NW) for j in range(_NK)]

        def do_wblock(wb, carry):
            lb, ob = carry
            for ke in (0, 1):
                emit_tile(
                    lb, lb + _NK * _C * ke, ob + _D - _NK * ke,
                    _D - _NK * ke, _valid_pairs(ke), False,
                )

            def interior(k, kc):
                rbk, obk = kc
                emit_tile(lb, rbk, obk, 0, all_pairs, True)
                return (rbk + _NK * _C, obk - _NK)

            lax.fori_loop(
                2, _NKB - 1, interior,
                (lb + 2 * _NK * _C, ob + _D - 2 * _NK),
            )
            for ke in (_NKB - 1, _NKB):
                emit_tile(
                    lb, lb + _NK * _C * ke, ob + _D - _NK * ke,
                    _D - _NK * ke, _valid_pairs(ke), False,
                )
            return (lb + _NW * _C, ob + _NW * _D)

        lax.fori_loop(0, _W // _NW, do_wblock, (jnp.int32(0), jnp.int32(0)))
        pltpu.sync_copy(out_v, out_hbm.at[row])
        return 0

    lax.fori_loop(0, rows_per, do_row, 0)


def kernel(left_features, right_features):
    b, h, w, c = left_features.shape
    l2 = left_features.reshape(b * h, w * c)
    r2 = right_features.reshape(b * h, w * c)
    mesh = plsc.VectorSubcoreMesh(
        core_axis_name="c", subcore_axis_name="s", num_cores=2, num_subcores=16
    )
    out = pl.kernel(
        _body,
        out_type=jax.ShapeDtypeStruct((b * h, w * _D), jnp.float32),
        mesh=mesh,
        compiler_params=pltpu.CompilerParams(needs_layout_passes=False),
        scratch_types=[
            pltpu.VMEM((_W * _C,), jnp.float32),
            pltpu.VMEM(((_PAD + _W) * _C,), jnp.float32),
            pltpu.VMEM((_W * _D,), jnp.float32),
            pltpu.SemaphoreType.DMA,
        ],
    )(l2, r2)
    return out.reshape(b, h, w, _D)


# speedup vs baseline: 1.1241x; 1.1241x over previous
import functools
import jax
import jax.numpy as jnp
from jax import lax
from jax.experimental import pallas as pl
from jax.experimental.pallas import tpu as pltpu
from jax.experimental.pallas import tpu_sc as plsc

_W = 128
_C = 96
_D = 48
_CCHUNKS = _C // 16
_PAD = _D
_NW = 8
_NK = 4
_NKB = (_D + _NK - 1) // _NK + 1


def _valid_pairs(k):
    return [
        (i, j)
        for i in range(_NW)
        for j in range(_NK)
        if 0 <= _D + i - _NK * k - j < _D
    ]


def _body(l_hbm, r_hbm, out_hbm, l_v, rpad_v, out_v, sem):
    n_cores = 2
    n_sub = 16
    wid = lax.axis_index("s") * n_cores + lax.axis_index("c")
    n_workers = n_cores * n_sub
    nh = l_hbm.shape[1]
    nrows = l_hbm.shape[0] * nh
    rows_per = nrows // n_workers

    zero16 = jnp.zeros((16,), jnp.float32)
    scale = jnp.float32(1.0 / _C)
    iota = lax.iota(jnp.int32, 16)
    xmask = {s: (iota & s) != 0 for s in (8, 4, 2, 1)}
    xperm = {s: iota ^ s for s in (8, 4, 2, 1)}

    def merge(a, b, s):
        sel_ab = jnp.where(xmask[s], b, a)
        sel_ba = jnp.where(xmask[s], a, b)
        return sel_ab + sel_ba.at[xperm[s]].get(mode="promise_in_bounds")

    half_i = jnp.right_shift(iota, 1)
    low_j = jnp.bitwise_and(iota, 1)
    dbase = half_i - low_j
    bitrev = [int(f"{t:04b}"[::-1], 2) for t in range(16)]

    def zero_row(i, _):
        for cb in range(_CCHUNKS):
            rpad_v[i, pl.ds(16 * cb, 16)] = zero16
        return 0

    lax.fori_loop(0, _PAD, zero_row, 0)

    def emit_tile(w0, k, koff, valid, full):
        used_i = sorted({i for i, _ in valid})
        used_j = sorted({j for _, j in valid})
        accs = {p: zero16 for p in valid}
        for cb in range(_CCHUNKS):
            lv = {i: l_v[w0 + i, pl.ds(16 * cb, 16)] for i in used_i}
            rv = {
                j: rpad_v[w0 + _NK * k + j, pl.ds(16 * cb, 16)]
                for j in used_j
            }
            for (i, j) in valid:
                accs[(i, j)] = accs[(i, j)] + lv[i] * rv[j]
        for g in range(2):
            leaves = []
            any_live = False
            for t in range(16):
                o = bitrev[t]
                p = (o >> 1, 2 * g + (o & 1))
                if p in accs:
                    leaves.append(accs[p])
                    any_live = True
                else:
                    leaves.append(zero16)
            if not any_live:
                continue
            vs = leaves
            for s in (8, 4, 2, 1):
                vs = [merge(vs[2 * m], vs[2 * m + 1], s) for m in range(len(vs) // 2)]
            tot = vs[0] * scale
            rows = half_i + w0
            cols = dbase + (koff - 2 * g)
            if full:
                plsc.store_scatter(out_v, [rows, cols], tot)
            else:
                mask = (cols >= 0) & (cols < _D)
                plsc.store_scatter(out_v, [rows, cols], tot, mask=mask)

    all_pairs = [(i, j) for i in range(_NW) for j in range(_NK)]

    def do_row(r, _):
        row = wid * rows_per + r
        bb = row // nh
        hh = row % nh
        pltpu.sync_copy(l_hbm.at[bb, hh], l_v)
        pltpu.sync_copy(r_hbm.at[bb, hh], rpad_v.at[pl.ds(_PAD, _W)])

        def do_wblock(wb, _):
            w0 = wb * _NW
            for ke in (0, 1):
                emit_tile(w0, ke, _D - _NK * ke, _valid_pairs(ke), False)

            def interior(k, koff):
                emit_tile(w0, k, koff, all_pairs, True)
                return koff - _NK

            lax.fori_loop(2, _NKB - 1, interior, jnp.int32(_D - 2 * _NK))
            for ke in (_NKB - 1, _NKB):
                emit_tile(w0, ke, _D - _NK * ke, _valid_pairs(ke), False)
            return 0

        lax.fori_loop(0, _W // _NW, do_wblock, 0)
        pltpu.sync_copy(out_v, out_hbm.at[bb, hh])
        return 0

    lax.fori_loop(0, rows_per, do_row, 0)


def kernel(left_features, right_features):
    b, h, w, c = left_features.shape
    mesh = plsc.VectorSubcoreMesh(
        core_axis_name="c", subcore_axis_name="s", num_cores=2, num_subcores=16
    )
    out = pl.kernel(
        _body,
        out_type=jax.ShapeDtypeStruct((b, h, w, _D), jnp.float32),
        mesh=mesh,
        compiler_params=pltpu.CompilerParams(needs_layout_passes=False),
        scratch_types=[
            pltpu.VMEM((_W, _C), jnp.float32),
            pltpu.VMEM((_PAD + _W, _C), jnp.float32),
            pltpu.VMEM((_W, _D), jnp.float32),
            pltpu.SemaphoreType.DMA,
        ],
    )(left_features, right_features)
    return out


# speedup vs baseline: 1.2266x; 1.0911x over previous
import functools
import jax
import jax.numpy as jnp
from jax import lax
from jax.experimental import pallas as pl
from jax.experimental.pallas import tpu as pltpu
from jax.experimental.pallas import tpu_sc as plsc

_W = 128
_C = 96
_D = 48
_CCHUNKS = _C // 16
_PAD = _D
_NW = 8
_NK = 4
_NKB = (_D + _NK - 1) // _NK + 1


def _valid_pairs(k):
    return [
        (i, j)
        for i in range(_NW)
        for j in range(_NK)
        if 0 <= _D + i - _NK * k - j < _D
    ]


def _body(l_hbm, r_hbm, out_hbm, l_v, rpad_v, out_v, sem_in0, sem_in1,
          sem_out0, sem_out1):
    n_cores = 2
    n_sub = 16
    wid = lax.axis_index("s") * n_cores + lax.axis_index("c")
    n_workers = n_cores * n_sub
    nh = l_hbm.shape[1]
    nrows = l_hbm.shape[0] * nh
    rows_per = nrows // n_workers
    sems_in = (sem_in0, sem_in1)
    sems_out = (sem_out0, sem_out1)

    zero16 = jnp.zeros((16,), jnp.float32)
    scale = jnp.float32(1.0 / _C)
    iota = lax.iota(jnp.int32, 16)
    xmask = {s: (iota & s) != 0 for s in (8, 4, 2, 1)}
    xperm = {s: iota ^ s for s in (8, 4, 2, 1)}

    def merge(a, b, s):
        sel_ab = jnp.where(xmask[s], b, a)
        sel_ba = jnp.where(xmask[s], a, b)
        return sel_ab + sel_ba.at[xperm[s]].get(mode="promise_in_bounds")

    half_i = jnp.right_shift(iota, 1)
    low_j = jnp.bitwise_and(iota, 1)
    dbase = half_i - low_j
    bitrev = [int(f"{t:04b}"[::-1], 2) for t in range(16)]

    def zero_row(i, _):
        for p in range(2):
            for cb in range(_CCHUNKS):
                rpad_v[p, i, pl.ds(16 * cb, 16)] = zero16
        return 0

    lax.fori_loop(0, _PAD, zero_row, 0)

    def issue_in(p, row):
        bb = row // nh
        hh = row % nh
        pltpu.async_copy(l_hbm.at[bb, hh], l_v.at[p], sems_in[p])
        pltpu.async_copy(
            r_hbm.at[bb, hh], rpad_v.at[p, pl.ds(_PAD, _W)], sems_in[p]
        )

    def wait_in(p):
        pltpu.make_async_copy(l_hbm.at[0, 0], l_v.at[p], sems_in[p]).wait()
        pltpu.make_async_copy(
            r_hbm.at[0, 0], rpad_v.at[p, pl.ds(_PAD, _W)], sems_in[p]
        ).wait()

    def issue_out(p, row):
        bb = row // nh
        hh = row % nh
        pltpu.async_copy(out_v.at[p], out_hbm.at[bb, hh], sems_out[p])

    def wait_out(p):
        pltpu.make_async_copy(out_v.at[p], out_hbm.at[0, 0], sems_out[p]).wait()

    def emit_tile(lv_r, rp_r, ov_r, w0, k, koff, valid, full):
        used_i = sorted({i for i, _ in valid})
        used_j = sorted({j for _, j in valid})
        accs = {p: zero16 for p in valid}
        for cb in range(_CCHUNKS):
            lv = {i: lv_r[w0 + i, pl.ds(16 * cb, 16)] for i in used_i}
            rv = {
                j: rp_r[w0 + _NK * k + j, pl.ds(16 * cb, 16)]
                for j in used_j
            }
            for (i, j) in valid:
                accs[(i, j)] = accs[(i, j)] + lv[i] * rv[j]
        for g in range(2):
            leaves = []
            any_live = False
            for t in range(16):
                o = bitrev[t]
                p = (o >> 1, 2 * g + (o & 1))
                if p in accs:
                    leaves.append(accs[p])
                    any_live = True
                else:
                    leaves.append(zero16)
            if not any_live:
                continue
            vs = leaves
            for s in (8, 4, 2, 1):
                vs = [merge(vs[2 * m], vs[2 * m + 1], s) for m in range(len(vs) // 2)]
            tot = vs[0] * scale
            rows = half_i + w0
            cols = dbase + (koff - 2 * g)
            if full:
                plsc.store_scatter(ov_r, [rows, cols], tot)
            else:
                mask = (cols >= 0) & (cols < _D)
                plsc.store_scatter(ov_r, [rows, cols], tot, mask=mask)

    all_pairs = [(i, j) for i in range(_NW) for j in range(_NK)]

    base_row = wid * rows_per
    issue_in(0, base_row)

    def do_row(r, _):
        row = base_row + r
        par = r % 2

        @pl.when(jnp.logical_and(r + 1 < rows_per, par == 0))
        def _():
            issue_in(1, row + 1)

        @pl.when(jnp.logical_and(r + 1 < rows_per, par == 1))
        def _():
            issue_in(0, row + 1)

        @pl.when(par == 0)
        def _():
            wait_in(0)

        @pl.when(par == 1)
        def _():
            wait_in(1)

        @pl.when(jnp.logical_and(r >= 2, par == 0))
        def _():
            wait_out(0)

        @pl.when(jnp.logical_and(r >= 2, par == 1))
        def _():
            wait_out(1)

        lv_r = l_v.at[par]
        rp_r = rpad_v.at[par]
        ov_r = out_v.at[par]

        def do_wblock(wb, _):
            w0 = wb * _NW
            for ke in (0, 1):
                emit_tile(
                    lv_r, rp_r, ov_r, w0, ke, _D - _NK * ke,
                    _valid_pairs(ke), False,
                )

            def interior(k, koff):
                emit_tile(lv_r, rp_r, ov_r, w0, k, koff, all_pairs, True)
                return koff - _NK

            lax.fori_loop(2, _NKB - 1, interior, jnp.int32(_D - 2 * _NK))
            for ke in (_NKB - 1, _NKB):
                emit_tile(
                    lv_r, rp_r, ov_r, w0, ke, _D - _NK * ke,
                    _valid_pairs(ke), False,
                )
            return 0

        lax.fori_loop(0, _W // _NW, do_wblock, 0)

        @pl.when(par == 0)
        def _():
            issue_out(0, row)

        @pl.when(par == 1)
        def _():
            issue_out(1, row)

        return 0

    lax.fori_loop(0, rows_per, do_row, 0)
    wait_out(0)
    wait_out(1)


def kernel(left_features, right_features):
    b, h, w, c = left_features.shape
    mesh = plsc.VectorSubcoreMesh(
        core_axis_name="c", subcore_axis_name="s", num_cores=2, num_subcores=16
    )
    out = pl.kernel(
        _body,
        out_type=jax.ShapeDtypeStruct((b, h, w, _D), jnp.float32),
        mesh=mesh,
        compiler_params=pltpu.CompilerParams(needs_layout_passes=False),
        scratch_types=[
            pltpu.VMEM((2, _W, _C), jnp.float32),
            pltpu.VMEM((2, _PAD + _W, _C), jnp.float32),
            pltpu.VMEM((2, _W, _D), jnp.float32),
            pltpu.SemaphoreType.DMA,
            pltpu.SemaphoreType.DMA,
            pltpu.SemaphoreType.DMA,
            pltpu.SemaphoreType.DMA,
        ],
    )(left_features, right_features)
    return out


# speedup vs baseline: 1.2292x; 1.0022x over previous
import functools
import jax
import jax.numpy as jnp
from jax import lax
from jax.experimental import pallas as pl
from jax.experimental.pallas import tpu as pltpu
from jax.experimental.pallas import tpu_sc as plsc

_W = 128
_C = 96
_D = 48
_CCHUNKS = _C // 16
_PAD = _D
_NW = 8
_NK = 4
_NKB = (_D + _NK - 1) // _NK + 1


def _valid_pairs(k):
    return [
        (i, j)
        for i in range(_NW)
        for j in range(_NK)
        if 0 <= _D + i - _NK * k - j < _D
    ]


def _body(l_hbm, r_hbm, out_hbm, l_v, rpad_v, out_v, sem_in0, sem_in1,
          sem_out0, sem_out1):
    n_cores = 2
    n_sub = 16
    wid = lax.axis_index("s") * n_cores + lax.axis_index("c")
    n_workers = n_cores * n_sub
    nh = l_hbm.shape[1]
    nrows = l_hbm.shape[0] * nh
    rows_per = nrows // n_workers
    sems_in = (sem_in0, sem_in1)
    sems_out = (sem_out0, sem_out1)

    zero16 = jnp.zeros((16,), jnp.float32)
    scale = jnp.float32(1.0 / _C)
    iota = lax.iota(jnp.int32, 16)
    xmask = {s: (iota & s) != 0 for s in (8, 4, 2, 1)}
    xperm = {s: iota ^ s for s in (8, 4, 2, 1)}

    def merge(a, b, s):
        sel_ab = jnp.where(xmask[s], b, a)
        sel_ba = jnp.where(xmask[s], a, b)
        return sel_ab + sel_ba.at[xperm[s]].get(mode="promise_in_bounds")

    half_i = jnp.right_shift(iota, 1)
    low_j = jnp.bitwise_and(iota, 1)
    dbase = half_i - low_j
    bitrev = [int(f"{t:04b}"[::-1], 2) for t in range(16)]

    def zero_row(i, _):
        for p in range(2):
            for cb in range(_CCHUNKS):
                rpad_v[p, i, pl.ds(16 * cb, 16)] = zero16
        return 0

    lax.fori_loop(0, _PAD, zero_row, 0)

    def issue_in(p, row):
        bb = row // nh
        hh = row % nh
        pltpu.async_copy(l_hbm.at[bb, hh], l_v.at[p], sems_in[p])
        pltpu.async_copy(
            r_hbm.at[bb, hh], rpad_v.at[p, pl.ds(_PAD, _W)], sems_in[p]
        )

    def wait_in(p):
        pltpu.make_async_copy(l_hbm.at[0, 0], l_v.at[p], sems_in[p]).wait()
        pltpu.make_async_copy(
            r_hbm.at[0, 0], rpad_v.at[p, pl.ds(_PAD, _W)], sems_in[p]
        ).wait()

    def issue_out(p, row):
        bb = row // nh
        hh = row % nh
        pltpu.async_copy(out_v.at[p], out_hbm.at[bb, hh], sems_out[p])

    def wait_out(p):
        pltpu.make_async_copy(out_v.at[p], out_hbm.at[0, 0], sems_out[p]).wait()

    def emit_tile(lv_r, rp_r, ov_r, w0, k, koff, valid, full):
        used_i = sorted({i for i, _ in valid})
        used_j = sorted({j for _, j in valid})
        accs = {p: zero16 for p in valid}
        for cb in range(_CCHUNKS):
            lv = {i: lv_r[w0 + i, pl.ds(16 * cb, 16)] for i in used_i}
            rv = {
                j: rp_r[w0 + _NK * k + j, pl.ds(16 * cb, 16)]
                for j in used_j
            }
            for (i, j) in valid:
                accs[(i, j)] = accs[(i, j)] + lv[i] * rv[j]
        for g in range(2):
            leaves = []
            any_live = False
            for t in range(16):
                o = bitrev[t]
                p = (o >> 1, 2 * g + (o & 1))
                if p in accs:
                    leaves.append(accs[p])
                    any_live = True
                else:
                    leaves.append(zero16)
            if not any_live:
                continue
            vs = leaves
            for s in (8, 4, 2, 1):
                vs = [merge(vs[2 * m], vs[2 * m + 1], s) for m in range(len(vs) // 2)]
            tot = vs[0] * scale
            rows = half_i + w0
            cols = dbase + (koff - 2 * g)
            if full:
                plsc.store_scatter(ov_r, [rows, cols], tot)
            else:
                mask = (cols >= 0) & (cols < _D)
                plsc.store_scatter(ov_r, [rows, cols], tot, mask=mask)

    all_pairs = [(i, j) for i in range(_NW) for j in range(_NK)]

    base_row = wid * rows_per
    issue_in(0, base_row)

    def do_row(r, _):
        row = base_row + r
        par = r % 2

        @pl.when(jnp.logical_and(r + 1 < rows_per, par == 0))
        def _():
            issue_in(1, row + 1)

        @pl.when(jnp.logical_and(r + 1 < rows_per, par == 1))
        def _():
            issue_in(0, row + 1)

        @pl.when(par == 0)
        def _():
            wait_in(0)

        @pl.when(par == 1)
        def _():
            wait_in(1)

        @pl.when(jnp.logical_and(r >= 2, par == 0))
        def _():
            wait_out(0)

        @pl.when(jnp.logical_and(r >= 2, par == 1))
        def _():
            wait_out(1)

        lv_r = l_v.at[par]
        rp_r = rpad_v.at[par]
        ov_r = out_v.at[par]

        def do_wblock(wb, _):
            w0 = wb * _NW
            for ke in (0, 1):
                emit_tile(
                    lv_r, rp_r, ov_r, w0, ke, _D - _NK * ke,
                    _valid_pairs(ke), False,
                )

            def interior(k, koff):
                emit_tile(lv_r, rp_r, ov_r, w0, k, koff, all_pairs, True)
                return koff - _NK

            lax.fori_loop(
                2, _NKB - 1, interior, jnp.int32(_D - 2 * _NK), unroll=2
            )
            for ke in (_NKB - 1, _NKB):
                emit_tile(
                    lv_r, rp_r, ov_r, w0, ke, _D - _NK * ke,
                    _valid_pairs(ke), False,
                )
            return 0

        lax.fori_loop(0, _W // _NW, do_wblock, 0)

        @pl.when(par == 0)
        def _():
            issue_out(0, row)

        @pl.when(par == 1)
        def _():
            issue_out(1, row)

        return 0

    lax.fori_loop(0, rows_per, do_row, 0)
    wait_out(0)
    wait_out(1)


def kernel(left_features, right_features):
    b, h, w, c = left_features.shape
    mesh = plsc.VectorSubcoreMesh(
        core_axis_name="c", subcore_axis_name="s", num_cores=2, num_subcores=16
    )
    out = pl.kernel(
        _body,
        out_type=jax.ShapeDtypeStruct((b, h, w, _D), jnp.float32),
        mesh=mesh,
        compiler_params=pltpu.CompilerParams(
            needs_layout_passes=False,
            disable_bounds_checks=True,
            disable_semaphore_checks=True,
        ),
        scratch_types=[
            pltpu.VMEM((2, _W, _C), jnp.float32),
            pltpu.VMEM((2, _PAD + _W, _C), jnp.float32),
            pltpu.VMEM((2, _W, _D), jnp.float32),
            pltpu.SemaphoreType.DMA,
            pltpu.SemaphoreType.DMA,
            pltpu.SemaphoreType.DMA,
            pltpu.SemaphoreType.DMA,
        ],
    )(left_features, right_features)
    return out
